# baseline (speedup 1.0000x reference)
import numpy as np
import jax
import jax.numpy as jnp
from jax.experimental import pallas as pl
from jax.experimental.pallas import tpu as pltpu

_PARTS = 8
_TWO_PI = np.float32(2.0 * np.pi)
_BIN_W = np.float32((2.0 * np.pi) / _PARTS)


def _body(x_ref, w1va_ref, b1va_ref, w2va_ref, b2va_ref,
          w1dd_ref, b1dd_ref, w2dd_ref, b2dd_ref, out_ref):
    x = x_ref[...]
    bc, n, _ = x.shape

    msum = jnp.sum(x, axis=2, keepdims=True)
    cat = jnp.concatenate(
        [x[:, :, 0:2], x[:, :, 38:40], msum], axis=2)
    t = jnp.swapaxes(cat, 1, 2)
    fx = t[:, 0, :]
    fy = t[:, 1, :]
    lx = t[:, 2, :]
    ly = t[:, 3, :]
    ms = t[:, 4, :]
    dx = lx - fx
    dy = ly - fy
    vel = jnp.sqrt(dx * dx + dy * dy)
    dis = jnp.sqrt(lx * lx + ly * ly)
    dire = jnp.arctan2(lx, ly) % _TWO_PI
    idx = (dire / _BIN_W).astype(jnp.int32)
    idx = jnp.where(ms != 0.0, idx, -1)

    velm_l, dism_l, dirm_l = [], [], []
    for ang in range(_PARTS):
        m = (idx == ang).astype(jnp.float32)
        cnt = jnp.sum(m, axis=1, keepdims=True) + 0.0001
        velm_l.append(jnp.sum(vel * m, axis=1, keepdims=True) / cnt)
        dism_l.append(jnp.sum(dis * m, axis=1, keepdims=True) / cnt)
        dirm_l.append(jnp.sum(dire * m, axis=1, keepdims=True) / cnt)
    velm = jnp.concatenate(velm_l, axis=1)
    dism = jnp.concatenate(dism_l, axis=1)
    dirm = jnp.concatenate(dirm_l, axis=1)

    w1va = w1va_ref[...]
    h_va = jnp.maximum(
        velm[:, :, None] * w1va[0][None, None, :]
        + b1va_ref[...][None, None, :], 0.0)
    h_va2 = jax.lax.dot_general(
        h_va.reshape(bc * _PARTS, 64), w2va_ref[...],
        (((1,), (0,)), ((), ())), preferred_element_type=jnp.float32)
    f_va = jnp.tanh(h_va2 + b2va_ref[...][None, :]).reshape(bc, _PARTS, 64)

    w1dd = w1dd_ref[...]
    h_dd = jnp.maximum(
        dism[:, :, None] * w1dd[0][None, None, :]
        + dirm[:, :, None] * w1dd[1][None, None, :]
        + b1dd_ref[...][None, None, :], 0.0)
    h_dd2 = jax.lax.dot_general(
        h_dd.reshape(bc * _PARTS, 64), w2dd_ref[...],
        (((1,), (0,)), ((), ())), preferred_element_type=jnp.float32)
    f_dd = jnp.tanh(h_dd2 + b2dd_ref[...][None, :]).reshape(bc, _PARTS, 64)

    fused = jnp.concatenate([f_va, f_dd], axis=2)
    pad = jnp.zeros((bc, 20 - _PARTS, 128), jnp.float32)
    out_ref[...] = jnp.concatenate([fused, pad], axis=1)


def kernel(trajs, nei_trajs, W1_va, b1_va, W2_va, b2_va, W1_dd, b1_dd, W2_dd, b2_dd):
    B, N, T, _ = nei_trajs.shape
    x = nei_trajs.reshape(B, N, T * 2)
    BC = 128
    grid = (B // BC,)

    def const_spec(a):
        nd = a.ndim
        return pl.BlockSpec(a.shape, lambda i: (0,) * nd)

    out = pl.pallas_call(
        _body,
        grid=grid,
        in_specs=[
            pl.BlockSpec((BC, N, T * 2), lambda i: (i, 0, 0)),
            const_spec(W1_va), const_spec(b1_va), const_spec(W2_va), const_spec(b2_va),
            const_spec(W1_dd), const_spec(b1_dd), const_spec(W2_dd), const_spec(b2_dd),
        ],
        out_specs=pl.BlockSpec((BC, 20, 128), lambda i: (i, 0, 0)),
        out_shape=jax.ShapeDtypeStruct((B, 20, 128), jnp.float32),
        compiler_params=pltpu.CompilerParams(
            dimension_semantics=("arbitrary",),
        ),
    )(x, W1_va, b1_va, W2_va, b2_va, W1_dd, b1_dd, W2_dd, b2_dd)
    return out


# speedup vs baseline: 1.6722x; 1.6722x over previous
import numpy as np
import jax
import jax.numpy as jnp
from jax import lax
from jax.experimental import pallas as pl
from jax.experimental.pallas import tpu as pltpu

_PARTS = 8
_TWO_PI = np.float32(2.0 * np.pi)
_BIN_W = np.float32((2.0 * np.pi) / _PARTS)


def _body(x_ref, w1va_ref, b1va_ref, w2va_ref, b2va_ref,
          w1dd_ref, b1dd_ref, w2dd_ref, b2dd_ref, out_ref):
    bc, n, rec = x_ref.shape
    xr = x_ref[...].reshape(bc * n, rec)

    r = lax.broadcasted_iota(jnp.int32, (8, rec), 1)
    c = lax.broadcasted_iota(jnp.int32, (8, rec), 0)
    one = jnp.float32(1.0)
    zero = jnp.float32(0.0)
    e0 = jnp.where(r == 0, one, zero)
    e1 = jnp.where(r == 1, one, zero)
    el2 = jnp.where(r == rec - 2, one, zero)
    el1 = jnp.where(r == rec - 1, one, zero)
    st = (jnp.where(c == 0, one, zero)
          + jnp.where(c == 1, el2, zero)
          + jnp.where(c == 2, el1, zero)
          + jnp.where(c == 3, el2 - e0, zero)
          + jnp.where(c == 4, el1 - e1, zero))

    f = lax.dot_general(st, xr, (((1,), (1,)), ((), ())),
                        preferred_element_type=jnp.float32)
    f = f.reshape(8, bc, n)
    ms = f[0]
    lx = f[1]
    ly = f[2]
    dx = f[3]
    dy = f[4]

    vel = jnp.sqrt(dx * dx + dy * dy)
    dis = jnp.sqrt(lx * lx + ly * ly)
    dire = jnp.arctan2(lx, ly)
    dire = jnp.where(dire < 0.0, dire + _TWO_PI, dire)
    idx = (dire / _BIN_W).astype(jnp.int32)
    idx = jnp.where(ms != 0.0, idx, -1)

    velm_l, dism_l, dirm_l = [], [], []
    for ang in range(_PARTS):
        m = (idx == ang).astype(jnp.float32)
        cnt = jnp.sum(m, axis=1, keepdims=True) + 0.0001
        velm_l.append(jnp.sum(vel * m, axis=1, keepdims=True) / cnt)
        dism_l.append(jnp.sum(dis * m, axis=1, keepdims=True) / cnt)
        dirm_l.append(jnp.sum(dire * m, axis=1, keepdims=True) / cnt)
    velm = jnp.concatenate(velm_l, axis=1)
    dism = jnp.concatenate(dism_l, axis=1)
    dirm = jnp.concatenate(dirm_l, axis=1)

    w1va = w1va_ref[...]
    h_va = jnp.maximum(
        velm[:, :, None] * w1va[0][None, None, :]
        + b1va_ref[...][None, None, :], 0.0)
    h_va2 = lax.dot_general(
        h_va.reshape(bc * _PARTS, 64), w2va_ref[...],
        (((1,), (0,)), ((), ())), preferred_element_type=jnp.float32)
    f_va = jnp.tanh(h_va2 + b2va_ref[...][None, :]).reshape(bc, _PARTS, 64)

    w1dd = w1dd_ref[...]
    h_dd = jnp.maximum(
        dism[:, :, None] * w1dd[0][None, None, :]
        + dirm[:, :, None] * w1dd[1][None, None, :]
        + b1dd_ref[...][None, None, :], 0.0)
    h_dd2 = lax.dot_general(
        h_dd.reshape(bc * _PARTS, 64), w2dd_ref[...],
        (((1,), (0,)), ((), ())), preferred_element_type=jnp.float32)
    f_dd = jnp.tanh(h_dd2 + b2dd_ref[...][None, :]).reshape(bc, _PARTS, 64)

    fused = jnp.concatenate([f_va, f_dd], axis=2)
    pad = jnp.zeros((bc, 20 - _PARTS, 128), jnp.float32)
    out_ref[...] = jnp.concatenate([fused, pad], axis=1)


def kernel(trajs, nei_trajs, W1_va, b1_va, W2_va, b2_va, W1_dd, b1_dd, W2_dd, b2_dd):
    B, N, T, _ = nei_trajs.shape
    REC = T * 2
    x = nei_trajs.reshape(B, N, REC)

    BC = 64

    def const_spec(a):
        nd = a.ndim
        return pl.BlockSpec(a.shape, lambda i: (0,) * nd)

    out = pl.pallas_call(
        _body,
        grid=(B // BC,),
        in_specs=[
            pl.BlockSpec((BC, N, REC), lambda i: (i, 0, 0)),
            const_spec(W1_va), const_spec(b1_va), const_spec(W2_va), const_spec(b2_va),
            const_spec(W1_dd), const_spec(b1_dd), const_spec(W2_dd), const_spec(b2_dd),
        ],
        out_specs=pl.BlockSpec((BC, 20, 128), lambda i: (i, 0, 0)),
        out_shape=jax.ShapeDtypeStruct((B, 20, 128), jnp.float32),
        compiler_params=pltpu.CompilerParams(
            dimension_semantics=("parallel",),
        ),
    )(x, W1_va, b1_va, W2_va, b2_va, W1_dd, b1_dd, W2_dd, b2_dd)
    return out


# speedup vs baseline: 1.8260x; 1.0919x over previous
import numpy as np
import jax
import jax.numpy as jnp
from jax import lax
from jax.experimental import pallas as pl
from jax.experimental.pallas import tpu as pltpu

_PARTS = 8
_TWO_PI = np.float32(2.0 * np.pi)
_BIN_W = np.float32((2.0 * np.pi) / _PARTS)


def _body(x_ref, w1va_ref, b1va_ref, w2va_ref, b2va_ref,
          w1dd_ref, b1dd_ref, w2dd_ref, b2dd_ref, out_ref):
    bc, n, rec = x_ref.shape
    xr = x_ref[...].reshape(bc * n, rec)

    r = lax.broadcasted_iota(jnp.int32, (8, rec), 1)
    c = lax.broadcasted_iota(jnp.int32, (8, rec), 0)
    one = jnp.float32(1.0)
    zero = jnp.float32(0.0)
    e0 = jnp.where(r == 0, one, zero)
    e1 = jnp.where(r == 1, one, zero)
    el2 = jnp.where(r == rec - 2, one, zero)
    el1 = jnp.where(r == rec - 1, one, zero)
    st = (jnp.where(c == 0, one, zero)
          + jnp.where(c == 1, el2, zero)
          + jnp.where(c == 2, el1, zero)
          + jnp.where(c == 3, el2 - e0, zero)
          + jnp.where(c == 4, el1 - e1, zero))

    f = lax.dot_general(st, xr, (((1,), (1,)), ((), ())),
                        preferred_element_type=jnp.float32)
    f = f.reshape(8, bc, n)
    ms = f[0]
    lx = f[1]
    ly = f[2]
    dx = f[3]
    dy = f[4]

    vel = jnp.sqrt(dx * dx + dy * dy)
    dis = jnp.sqrt(lx * lx + ly * ly)
    dire = jnp.arctan2(lx, ly)
    dire = jnp.where(dire < 0.0, dire + _TWO_PI, dire)
    idx = (dire / _BIN_W).astype(jnp.int32)
    idx = jnp.where(ms != 0.0, idx, -1)

    velm_l, dism_l, dirm_l = [], [], []
    for ang in range(_PARTS):
        m = (idx == ang).astype(jnp.float32)
        cnt = jnp.sum(m, axis=1, keepdims=True) + 0.0001
        velm_l.append(jnp.sum(vel * m, axis=1, keepdims=True) / cnt)
        dism_l.append(jnp.sum(dis * m, axis=1, keepdims=True) / cnt)
        dirm_l.append(jnp.sum(dire * m, axis=1, keepdims=True) / cnt)
    velm = jnp.concatenate(velm_l, axis=1)
    dism = jnp.concatenate(dism_l, axis=1)
    dirm = jnp.concatenate(dirm_l, axis=1)

    w1va = w1va_ref[...]
    h_va = jnp.maximum(
        velm[:, :, None] * w1va[0][None, None, :]
        + b1va_ref[...][None, None, :], 0.0)
    h_va2 = lax.dot_general(
        h_va.reshape(bc * _PARTS, 64), w2va_ref[...],
        (((1,), (0,)), ((), ())), preferred_element_type=jnp.float32)
    f_va = jnp.tanh(h_va2 + b2va_ref[...][None, :]).reshape(bc, _PARTS, 64)

    w1dd = w1dd_ref[...]
    h_dd = jnp.maximum(
        dism[:, :, None] * w1dd[0][None, None, :]
        + dirm[:, :, None] * w1dd[1][None, None, :]
        + b1dd_ref[...][None, None, :], 0.0)
    h_dd2 = lax.dot_general(
        h_dd.reshape(bc * _PARTS, 64), w2dd_ref[...],
        (((1,), (0,)), ((), ())), preferred_element_type=jnp.float32)
    f_dd = jnp.tanh(h_dd2 + b2dd_ref[...][None, :]).reshape(bc, _PARTS, 64)

    fused = jnp.concatenate([f_va, f_dd], axis=2)
    pad = jnp.zeros((bc, 20 - _PARTS, 128), jnp.float32)
    out_ref[...] = jnp.concatenate([fused, pad], axis=1)


def kernel(trajs, nei_trajs, W1_va, b1_va, W2_va, b2_va, W1_dd, b1_dd, W2_dd, b2_dd):
    B, N, T, _ = nei_trajs.shape
    REC = T * 2
    x = nei_trajs.reshape(B, N, REC)

    BC = 256

    def const_spec(a):
        nd = a.ndim
        return pl.BlockSpec(a.shape, lambda i: (0,) * nd)

    out = pl.pallas_call(
        _body,
        grid=(B // BC,),
        in_specs=[
            pl.BlockSpec((BC, N, REC), lambda i: (i, 0, 0)),
            const_spec(W1_va), const_spec(b1_va), const_spec(W2_va), const_spec(b2_va),
            const_spec(W1_dd), const_spec(b1_dd), const_spec(W2_dd), const_spec(b2_dd),
        ],
        out_specs=pl.BlockSpec((BC, 20, 128), lambda i: (i, 0, 0)),
        out_shape=jax.ShapeDtypeStruct((B, 20, 128), jnp.float32),
        compiler_params=pltpu.CompilerParams(
            dimension_semantics=("parallel",),
        ),
    )(x, W1_va, b1_va, W2_va, b2_va, W1_dd, b1_dd, W2_dd, b2_dd)
    return out
